# Initial kernel scaffold; baseline (speedup 1.0000x reference)
import jax
import jax.numpy as jnp
from jax.experimental import pallas as pl


def kernel(x, table):
    raise NotImplementedError("write your pallas kernel here")


# speedup vs baseline: 3.2632x; 3.2632x over previous
import functools

import jax
import jax.numpy as jnp
from jax import lax
from jax.experimental import pallas as pl
from jax.experimental.pallas import tpu as pltpu
from jax.experimental.pallas import tpu_sc as plsc

NC, NS, LANES = 2, 16, 16
NW = NC * NS
VOCAB, D = 100000, 64
B, L = 4096, 200
TOK = B * L
PER_W = TOK // NW
CHUNK = 512
IDXW = 128
K = CHUNK // IDXW
NCHUNKS = PER_W // CHUNK
IDX_ROWS_PER_W = PER_W // IDXW

_mesh = plsc.VectorSubcoreMesh(
    core_axis_name="c", subcore_axis_name="s", num_cores=NC, num_subcores=NS
)


def _sc_body(table_hbm, idx_hbm, ratio_hbm, out_hbm, idx_v, ratio_v, rows_v, gsem):
    wid = lax.axis_index("s") * NC + lax.axis_index("c")

    def chunk_body(g, carry):
        row0 = wid * IDX_ROWS_PER_W + g * K
        tok0 = wid * PER_W + g * CHUNK
        pltpu.sync_copy(idx_hbm.at[pl.ds(row0, K)], idx_v)
        pltpu.sync_copy(ratio_hbm.at[pl.ds(tok0, CHUNK)], ratio_v)
        descs = [
            pltpu.async_copy(
                table_hbm.at[idx_v.at[j]],
                rows_v.at[pl.ds(j * IDXW, IDXW)],
                gsem,
            )
            for j in range(K)
        ]
        for d in descs:
            d.wait()

        def mul_body(t, c):
            base = t * LANES
            rv = ratio_v[pl.ds(base, LANES)] * 8.0
            for k in range(LANES):
                rvec = jnp.full((LANES,), rv[k], jnp.float32)
                for j in range(D // LANES):
                    sl = pl.ds(j * LANES, LANES)
                    rows_v[base + k, sl] = rows_v[base + k, sl] * rvec
            return c

        lax.fori_loop(0, CHUNK // LANES, mul_body, 0)
        pltpu.sync_copy(rows_v, out_hbm.at[pl.ds(tok0, CHUNK)])
        return carry

    lax.fori_loop(0, NCHUNKS, chunk_body, 0)


_sc_call = functools.partial(
    pl.kernel,
    out_type=jax.ShapeDtypeStruct((TOK, D), jnp.float32),
    mesh=_mesh,
    compiler_params=pltpu.CompilerParams(use_tc_tiling_on_sc=False),
    scratch_types=[
        pltpu.VMEM((K, IDXW), jnp.int32),
        pltpu.VMEM((CHUNK,), jnp.float32),
        pltpu.VMEM((CHUNK, D), jnp.float32),
        pltpu.SemaphoreType.DMA,
    ],
)(_sc_body)


def kernel(x, table):
    words = x[:, 0, :].reshape(TOK).astype(jnp.int32)
    ratio = x[:, 1, :].reshape(TOK)
    idx2d = words.reshape(TOK // IDXW, IDXW)
    out = _sc_call(table, idx2d, ratio)
    return out.reshape(B, L, D)
